# baseline (speedup 1.0000x reference)
import functools

import jax
import jax.numpy as jnp
from jax import lax
from jax.experimental import pallas as pl
from jax.experimental.pallas import tpu as pltpu
from jax.experimental.pallas import tpu_sc as plsc

N = 10000
E = 320000
G = 256
D = 128
NUM_LAYERS = 5
BLK = 400
NB = N // BLK

SC_CORES = 2
SC_SUBC = 16
NTILES = SC_CORES * SC_SUBC
CHUNK = 128
NCT = 80
EPAD = NTILES * NCT * CHUNK
PCHUNK = 640
NPAD = SC_SUBC * PCHUNK
ROWS_PER_TILE = PCHUNK
NBUF = 4

_sc_mesh = plsc.VectorSubcoreMesh(
    core_axis_name="c", subcore_axis_name="s",
    num_cores=SC_CORES, num_subcores=SC_SUBC)

_Z16 = functools.partial(jnp.zeros, (16,), jnp.float32)

_PIECES = ((0, 128), (128, 128), (256, 128), (384, 128), (512, 128))


@functools.partial(
    pl.kernel,
    out_type=jax.ShapeDtypeStruct((SC_CORES, NPAD, D), jnp.float32),
    mesh=_sc_mesh,
    scratch_types=[
        pltpu.VMEM_SHARED((NPAD, D), jnp.float32),
        pltpu.VMEM((2, CHUNK), jnp.int32),
        pltpu.VMEM((2, CHUNK), jnp.int32),
        pltpu.VMEM((2, CHUNK, D), jnp.float32),
        pltpu.SemaphoreType.DMA,
        pltpu.SemaphoreType.DMA,
    ],
)
def _agg_sc(h_hbm, src_hbm, dst_hbm, out_hbm, acc, sidx, didx, rows, g0, g1):
    cid = lax.axis_index("c")
    sid = lax.axis_index("s")
    w = cid * SC_SUBC + sid

    def _zrow(i, carry):
        for j in range(8):
            rows[0, i, 16 * j:16 * j + 16] = _Z16()
        return carry

    lax.fori_loop(0, CHUNK, _zrow, None)
    off = sid * ROWS_PER_TILE
    for k, n in _PIECES:
        pltpu.sync_copy(rows.at[0, pl.ds(0, n)], acc.at[pl.ds(off + k, n)])
    plsc.subcore_barrier()

    def body(t, carry):
        base = (w * NCT + 2 * t) * CHUNK
        pltpu.sync_copy(src_hbm.at[pl.ds(base, CHUNK)], sidx.at[0])
        pltpu.sync_copy(dst_hbm.at[pl.ds(base, CHUNK)], didx.at[0])
        pltpu.sync_copy(h_hbm.at[sidx.at[0]], rows.at[0])
        pltpu.sync_copy(rows.at[0], acc.at[didx.at[0]], add=True)
        pltpu.sync_copy(src_hbm.at[pl.ds(base + CHUNK, CHUNK)], sidx.at[1])
        pltpu.sync_copy(dst_hbm.at[pl.ds(base + CHUNK, CHUNK)], didx.at[1])
        pltpu.sync_copy(h_hbm.at[sidx.at[1]], rows.at[1])
        pltpu.sync_copy(rows.at[1], acc.at[didx.at[1]], add=True)
        return carry

    lax.fori_loop(0, NCT // 2, body, None)
    plsc.subcore_barrier()

    for k, n in _PIECES:
        pltpu.sync_copy(acc.at[pl.ds(off + k, n)], rows.at[0, pl.ds(0, n)])
        pltpu.sync_copy(rows.at[0, pl.ds(0, n)], out_hbm.at[cid, pl.ds(off + k, n)])


def _ln(z, g, b):
    mu = jnp.mean(z, axis=-1, keepdims=True)
    d = z - mu
    var = jnp.mean(d * d, axis=-1, keepdims=True)
    return d * lax.rsqrt(var + 1e-5) * g + b


def _embed_body(x_ref, emb_ref, out_ref):
    xb = x_ref[0, 0, :]
    ids = lax.broadcasted_iota(jnp.int32, (BLK, 32), 1)
    oh = (xb[:, None] == ids).astype(jnp.float32)
    out_ref[...] = jnp.dot(oh, emb_ref[...], preferred_element_type=jnp.float32)


_embed_call = pl.pallas_call(
    _embed_body,
    grid=(NB,),
    in_specs=[
        pl.BlockSpec((1, 1, BLK), lambda i: (i, 0, 0)),
        pl.BlockSpec((32, D), lambda i: (0, 0)),
    ],
    out_specs=pl.BlockSpec((BLK, D), lambda i: (i, 0)),
    out_shape=jax.ShapeDtypeStruct((N, D), jnp.float32),
)


def _mlp_body(h_ref, a_ref, w1_ref, b1_ref, g1_ref, e1_ref,
              w2_ref, b2_ref, g2_ref, e2_ref, out_ref):
    z = h_ref[...] + a_ref[0] + a_ref[1]
    z = jnp.dot(z, w1_ref[...], preferred_element_type=jnp.float32) + b1_ref[...]
    z = _ln(z, g1_ref[...], e1_ref[...])
    z = jnp.maximum(z, 0.0)
    h2 = jnp.dot(z, w2_ref[...], preferred_element_type=jnp.float32) + b2_ref[...]
    h2 = _ln(h2, g2_ref[...], e2_ref[...])
    out_ref[...] = jnp.maximum(h2, 0.0)


def _last_body(h_ref, a_ref, w1_ref, b1_ref, g1_ref, e1_ref,
               w2_ref, b2_ref, aggw_ref, batch_ref, pooled_ref):
    z = h_ref[...] + a_ref[0] + a_ref[1]
    z = jnp.dot(z, w1_ref[...], preferred_element_type=jnp.float32) + b1_ref[...]
    z = _ln(z, g1_ref[...], e1_ref[...])
    z = jnp.maximum(z, 0.0)
    h2 = jnp.dot(z, w2_ref[...], preferred_element_type=jnp.float32) + b2_ref[...]
    y = jnp.dot(h2, aggw_ref[...], preferred_element_type=jnp.float32)
    bb = batch_ref[0, 0, :]
    gids = lax.broadcasted_iota(jnp.int32, (G, BLK), 0)
    ohT = (gids == bb[None, :]).astype(jnp.float32)
    part = jnp.dot(ohT, y, preferred_element_type=jnp.float32)
    i = pl.program_id(0)

    @pl.when(i == 0)
    def _init():
        pooled_ref[...] = part

    @pl.when(i > 0)
    def _acc():
        pooled_ref[...] += part


_row = lambda i: (i, 0)
_whole = lambda i: (0, 0)
_HB = pl.BlockSpec((BLK, D), _row)
_AB = pl.BlockSpec((SC_CORES, BLK, D), lambda i: (0, i, 0))
_WB = pl.BlockSpec((D, D), _whole)
_VB = pl.BlockSpec((1, D), _whole)

_mlp_call = pl.pallas_call(
    _mlp_body,
    grid=(NB,),
    in_specs=[_HB, _AB, _WB, _VB, _VB, _VB, _WB, _VB, _VB, _VB],
    out_specs=_HB,
    out_shape=jax.ShapeDtypeStruct((N, D), jnp.float32),
)

_last_call = pl.pallas_call(
    _last_body,
    grid=(NB,),
    in_specs=[_HB, _AB, _WB, _VB, _VB, _VB, _WB, _VB,
              pl.BlockSpec((D, 1), _whole),
              pl.BlockSpec((1, 1, BLK), lambda i: (i, 0, 0))],
    out_specs=pl.BlockSpec((G, 1), _whole),
    out_shape=jax.ShapeDtypeStruct((G, 1), jnp.float32),
)


def kernel(x, edge_index, batch, params):
    pad = EPAD - E
    src = jnp.concatenate([edge_index[0].astype(jnp.int32),
                           jnp.zeros((pad,), jnp.int32)])
    dst = jnp.concatenate([edge_index[1].astype(jnp.int32),
                           N + (jnp.arange(pad, dtype=jnp.int32) % (NPAD - N))])
    emb = jnp.zeros((32, D), jnp.float32).at[:22].set(params['embed'])
    x3 = x.astype(jnp.int32).reshape(NB, 1, BLK)
    h = _embed_call(x3, emb)

    def v(p):
        return p.reshape(1, D)

    for l in range(NUM_LAYERS):
        agg2 = _agg_sc(h, src, dst)
        if l < NUM_LAYERS - 1:
            h = _mlp_call(h, agg2,
                          params[f'l{l}_W1'], v(params[f'l{l}_b1']),
                          v(params[f'l{l}_mlp_ln_g']), v(params[f'l{l}_mlp_ln_b']),
                          params[f'l{l}_W2'], v(params[f'l{l}_b2']),
                          v(params[f'l{l}_out_ln_g']), v(params[f'l{l}_out_ln_b']))
        else:
            b3 = batch.astype(jnp.int32).reshape(NB, 1, BLK)
            pooled = _last_call(h, agg2,
                                params[f'l{l}_W1'], v(params[f'l{l}_b1']),
                                v(params[f'l{l}_mlp_ln_g']), v(params[f'l{l}_mlp_ln_b']),
                                params[f'l{l}_W2'], v(params[f'l{l}_b2']),
                                params['agg_W'], b3)

    return pooled + params['agg_b']


# speedup vs baseline: 2.1334x; 2.1334x over previous
import functools

import jax
import jax.numpy as jnp
from jax import lax
from jax.experimental import pallas as pl
from jax.experimental.pallas import tpu as pltpu
from jax.experimental.pallas import tpu_sc as plsc

N = 10000
E = 320000
G = 256
D = 128
NUM_LAYERS = 5
BLK = 400
NB = N // BLK

SC_CORES = 2
SC_SUBC = 16
NTILES = SC_CORES * SC_SUBC
CHUNK = 128
NCHUNKS = E // CHUNK
PCHUNK = 640
NPAD = SC_SUBC * PCHUNK
ROWS_PER_TILE = PCHUNK
NBUF = 4

_sc_mesh = plsc.VectorSubcoreMesh(
    core_axis_name="c", subcore_axis_name="s",
    num_cores=SC_CORES, num_subcores=SC_SUBC)

_Z16 = functools.partial(jnp.zeros, (16,), jnp.float32)

_PIECES = ((0, 128), (128, 128), (256, 128), (384, 128), (512, 128))


@functools.partial(
    pl.kernel,
    out_type=jax.ShapeDtypeStruct((SC_CORES, NPAD, D), jnp.float32),
    mesh=_sc_mesh,
    scratch_types=[
        pltpu.VMEM_SHARED((NPAD, D), jnp.float32),
        pltpu.VMEM((2, CHUNK), jnp.int32),
        pltpu.VMEM((2, CHUNK), jnp.int32),
        pltpu.VMEM((2, CHUNK, D), jnp.float32),
    ],
)
def _agg_sc(h_hbm, src_hbm, dst_hbm, out_hbm, acc, sidx, didx, rows):
    cid = lax.axis_index("c")
    sid = lax.axis_index("s")
    w = cid * SC_SUBC + sid

    def _zrow(i, carry):
        for j in range(8):
            rows[0, i, 16 * j:16 * j + 16] = _Z16()
        return carry

    lax.fori_loop(0, CHUNK, _zrow, None)
    off = sid * ROWS_PER_TILE
    for k, n in _PIECES:
        pltpu.sync_copy(rows.at[0, pl.ds(0, n)], acc.at[pl.ds(off + k, n)])
    plsc.subcore_barrier()

    rem = NCHUNKS % NTILES
    nc = jnp.where(w < rem, NCHUNKS // NTILES + 1, NCHUNKS // NTILES)

    def body(k, carry):
        base = (w + k * NTILES) * CHUNK
        pltpu.sync_copy(src_hbm.at[pl.ds(base, CHUNK)], sidx.at[0])
        pltpu.sync_copy(dst_hbm.at[pl.ds(base, CHUNK)], didx.at[0])
        pltpu.sync_copy(h_hbm.at[sidx.at[0]], rows.at[0])
        pltpu.sync_copy(rows.at[0], acc.at[didx.at[0]], add=True)
        return carry

    lax.fori_loop(0, nc, body, None)
    plsc.subcore_barrier()

    for k, n in _PIECES:
        pltpu.sync_copy(acc.at[pl.ds(off + k, n)], rows.at[0, pl.ds(0, n)])
        pltpu.sync_copy(rows.at[0, pl.ds(0, n)], out_hbm.at[cid, pl.ds(off + k, n)])


def _ln(z, g, b):
    mu = jnp.mean(z, axis=-1, keepdims=True)
    d = z - mu
    var = jnp.mean(d * d, axis=-1, keepdims=True)
    return d * lax.rsqrt(var + 1e-5) * g + b


def _embed_body(x_ref, emb_ref, out_ref):
    xb = x_ref[0, 0, :]
    ids = lax.broadcasted_iota(jnp.int32, (BLK, 32), 1)
    oh = (xb[:, None] == ids).astype(jnp.float32)
    out_ref[...] = jnp.dot(oh, emb_ref[...], preferred_element_type=jnp.float32)


_embed_call = pl.pallas_call(
    _embed_body,
    grid=(NB,),
    in_specs=[
        pl.BlockSpec((1, 1, BLK), lambda i: (i, 0, 0)),
        pl.BlockSpec((32, D), lambda i: (0, 0)),
    ],
    out_specs=pl.BlockSpec((BLK, D), lambda i: (i, 0)),
    out_shape=jax.ShapeDtypeStruct((N, D), jnp.float32),
)


def _mlp_body(h_ref, a_ref, w1_ref, b1_ref, g1_ref, e1_ref,
              w2_ref, b2_ref, g2_ref, e2_ref, out_ref):
    z = h_ref[...] + a_ref[0] + a_ref[1]
    z = jnp.dot(z, w1_ref[...], preferred_element_type=jnp.float32) + b1_ref[...]
    z = _ln(z, g1_ref[...], e1_ref[...])
    z = jnp.maximum(z, 0.0)
    h2 = jnp.dot(z, w2_ref[...], preferred_element_type=jnp.float32) + b2_ref[...]
    h2 = _ln(h2, g2_ref[...], e2_ref[...])
    out_ref[...] = jnp.maximum(h2, 0.0)


def _last_body(h_ref, a_ref, w1_ref, b1_ref, g1_ref, e1_ref,
               w2_ref, b2_ref, aggw_ref, batch_ref, pooled_ref):
    z = h_ref[...] + a_ref[0] + a_ref[1]
    z = jnp.dot(z, w1_ref[...], preferred_element_type=jnp.float32) + b1_ref[...]
    z = _ln(z, g1_ref[...], e1_ref[...])
    z = jnp.maximum(z, 0.0)
    h2 = jnp.dot(z, w2_ref[...], preferred_element_type=jnp.float32) + b2_ref[...]
    y = jnp.dot(h2, aggw_ref[...], preferred_element_type=jnp.float32)
    bb = batch_ref[0, 0, :]
    gids = lax.broadcasted_iota(jnp.int32, (G, BLK), 0)
    ohT = (gids == bb[None, :]).astype(jnp.float32)
    part = jnp.dot(ohT, y, preferred_element_type=jnp.float32)
    i = pl.program_id(0)

    @pl.when(i == 0)
    def _init():
        pooled_ref[...] = part

    @pl.when(i > 0)
    def _acc():
        pooled_ref[...] += part


_row = lambda i: (i, 0)
_whole = lambda i: (0, 0)
_HB = pl.BlockSpec((BLK, D), _row)
_AB = pl.BlockSpec((SC_CORES, BLK, D), lambda i: (0, i, 0))
_WB = pl.BlockSpec((D, D), _whole)
_VB = pl.BlockSpec((1, D), _whole)

_mlp_call = pl.pallas_call(
    _mlp_body,
    grid=(NB,),
    in_specs=[_HB, _AB, _WB, _VB, _VB, _VB, _WB, _VB, _VB, _VB],
    out_specs=_HB,
    out_shape=jax.ShapeDtypeStruct((N, D), jnp.float32),
)

_last_call = pl.pallas_call(
    _last_body,
    grid=(NB,),
    in_specs=[_HB, _AB, _WB, _VB, _VB, _VB, _WB, _VB,
              pl.BlockSpec((D, 1), _whole),
              pl.BlockSpec((1, 1, BLK), lambda i: (i, 0, 0))],
    out_specs=pl.BlockSpec((G, 1), _whole),
    out_shape=jax.ShapeDtypeStruct((G, 1), jnp.float32),
)


def kernel(x, edge_index, batch, params):
    src = edge_index[0].astype(jnp.int32)
    dst = edge_index[1].astype(jnp.int32)
    emb = jnp.zeros((32, D), jnp.float32).at[:22].set(params['embed'])
    x3 = x.astype(jnp.int32).reshape(NB, 1, BLK)
    h = _embed_call(x3, emb)

    def v(p):
        return p.reshape(1, D)

    for l in range(NUM_LAYERS):
        agg2 = _agg_sc(h, src, dst)
        if l < NUM_LAYERS - 1:
            h = _mlp_call(h, agg2,
                          params[f'l{l}_W1'], v(params[f'l{l}_b1']),
                          v(params[f'l{l}_mlp_ln_g']), v(params[f'l{l}_mlp_ln_b']),
                          params[f'l{l}_W2'], v(params[f'l{l}_b2']),
                          v(params[f'l{l}_out_ln_g']), v(params[f'l{l}_out_ln_b']))
        else:
            b3 = batch.astype(jnp.int32).reshape(NB, 1, BLK)
            pooled = _last_call(h, agg2,
                                params[f'l{l}_W1'], v(params[f'l{l}_b1']),
                                v(params[f'l{l}_mlp_ln_g']), v(params[f'l{l}_mlp_ln_b']),
                                params[f'l{l}_W2'], v(params[f'l{l}_b2']),
                                params['agg_W'], b3)

    return pooled + params['agg_b']


# speedup vs baseline: 3.0869x; 1.4470x over previous
import functools

import jax
import jax.numpy as jnp
from jax import lax
from jax.experimental import pallas as pl
from jax.experimental.pallas import tpu as pltpu
from jax.experimental.pallas import tpu_sc as plsc

N = 10000
E = 320000
G = 256
D = 128
NUM_LAYERS = 5
BLK = 400
NB = N // BLK

SC_CORES = 2
SC_SUBC = 16
NTILES = SC_CORES * SC_SUBC
CHUNK = 128
NCHUNKS = E // CHUNK
PCHUNK = 640
NPAD = SC_SUBC * PCHUNK
ROWS_PER_TILE = PCHUNK
NBUF = 4

_sc_mesh = plsc.VectorSubcoreMesh(
    core_axis_name="c", subcore_axis_name="s",
    num_cores=SC_CORES, num_subcores=SC_SUBC)

_Z16 = functools.partial(jnp.zeros, (16,), jnp.float32)

_PIECES = ((0, 128), (128, 128), (256, 128), (384, 128), (512, 128))


@functools.partial(
    pl.kernel,
    out_type=jax.ShapeDtypeStruct((SC_CORES, NPAD, D), jnp.float32),
    mesh=_sc_mesh,
    scratch_types=[
        pltpu.VMEM_SHARED((NPAD, D), jnp.float32),
        pltpu.VMEM((2, CHUNK), jnp.int32),
        pltpu.VMEM((2, CHUNK), jnp.int32),
        pltpu.VMEM((2, CHUNK, D), jnp.float32),
        pltpu.SemaphoreType.DMA,
        pltpu.SemaphoreType.DMA,
    ],
)
def _agg_sc(h_hbm, src_hbm, dst_hbm, out_hbm, acc, sidx, didx, rows, g0, g1):
    cid = lax.axis_index("c")
    sid = lax.axis_index("s")
    w = cid * SC_SUBC + sid

    def _zrow(i, carry):
        for j in range(8):
            rows[0, i, 16 * j:16 * j + 16] = _Z16()
        return carry

    lax.fori_loop(0, CHUNK, _zrow, None)
    off = sid * ROWS_PER_TILE
    for k, n in _PIECES:
        pltpu.sync_copy(rows.at[0, pl.ds(0, n)], acc.at[pl.ds(off + k, n)])
    plsc.subcore_barrier()

    npairs = NCHUNKS // 2
    rem = npairs % NTILES
    nc = jnp.where(w < rem, npairs // NTILES + 1, npairs // NTILES)

    def body(t, carry):
        base = (2 * w + 2 * NTILES * t) * CHUNK
        pltpu.sync_copy(src_hbm.at[pl.ds(base, CHUNK)], sidx.at[0])
        pltpu.async_copy(h_hbm.at[sidx.at[0]], rows.at[0], g0)
        pltpu.sync_copy(src_hbm.at[pl.ds(base + CHUNK, CHUNK)], sidx.at[1])
        pltpu.async_copy(h_hbm.at[sidx.at[1]], rows.at[1], g1)
        pltpu.sync_copy(dst_hbm.at[pl.ds(base, CHUNK)], didx.at[0])
        pltpu.sync_copy(dst_hbm.at[pl.ds(base + CHUNK, CHUNK)], didx.at[1])
        pltpu.make_async_copy(h_hbm.at[pl.ds(0, CHUNK)], rows.at[0], g0).wait()
        pltpu.sync_copy(rows.at[0], acc.at[didx.at[0]], add=True)
        pltpu.make_async_copy(h_hbm.at[pl.ds(0, CHUNK)], rows.at[1], g1).wait()
        pltpu.sync_copy(rows.at[1], acc.at[didx.at[1]], add=True)
        return carry

    lax.fori_loop(0, nc, body, None)
    plsc.subcore_barrier()

    for k, n in _PIECES:
        pltpu.sync_copy(acc.at[pl.ds(off + k, n)], rows.at[0, pl.ds(0, n)])
        pltpu.sync_copy(rows.at[0, pl.ds(0, n)], out_hbm.at[cid, pl.ds(off + k, n)])


def _ln(z, g, b):
    mu = jnp.mean(z, axis=-1, keepdims=True)
    d = z - mu
    var = jnp.mean(d * d, axis=-1, keepdims=True)
    return d * lax.rsqrt(var + 1e-5) * g + b


def _embed_body(x_ref, emb_ref, out_ref):
    xb = x_ref[0, 0, :]
    ids = lax.broadcasted_iota(jnp.int32, (BLK, 32), 1)
    oh = (xb[:, None] == ids).astype(jnp.float32)
    out_ref[...] = jnp.dot(oh, emb_ref[...], preferred_element_type=jnp.float32)


_embed_call = pl.pallas_call(
    _embed_body,
    grid=(NB,),
    in_specs=[
        pl.BlockSpec((1, 1, BLK), lambda i: (i, 0, 0)),
        pl.BlockSpec((32, D), lambda i: (0, 0)),
    ],
    out_specs=pl.BlockSpec((BLK, D), lambda i: (i, 0)),
    out_shape=jax.ShapeDtypeStruct((N, D), jnp.float32),
)


def _mlp_body(h_ref, a_ref, w1_ref, b1_ref, g1_ref, e1_ref,
              w2_ref, b2_ref, g2_ref, e2_ref, out_ref):
    z = h_ref[...] + a_ref[0] + a_ref[1]
    z = jnp.dot(z, w1_ref[...], preferred_element_type=jnp.float32) + b1_ref[...]
    z = _ln(z, g1_ref[...], e1_ref[...])
    z = jnp.maximum(z, 0.0)
    h2 = jnp.dot(z, w2_ref[...], preferred_element_type=jnp.float32) + b2_ref[...]
    h2 = _ln(h2, g2_ref[...], e2_ref[...])
    out_ref[...] = jnp.maximum(h2, 0.0)


def _last_body(h_ref, a_ref, w1_ref, b1_ref, g1_ref, e1_ref,
               w2_ref, b2_ref, aggw_ref, batch_ref, pooled_ref):
    z = h_ref[...] + a_ref[0] + a_ref[1]
    z = jnp.dot(z, w1_ref[...], preferred_element_type=jnp.float32) + b1_ref[...]
    z = _ln(z, g1_ref[...], e1_ref[...])
    z = jnp.maximum(z, 0.0)
    h2 = jnp.dot(z, w2_ref[...], preferred_element_type=jnp.float32) + b2_ref[...]
    y = jnp.dot(h2, aggw_ref[...], preferred_element_type=jnp.float32)
    bb = batch_ref[0, 0, :]
    gids = lax.broadcasted_iota(jnp.int32, (G, BLK), 0)
    ohT = (gids == bb[None, :]).astype(jnp.float32)
    part = jnp.dot(ohT, y, preferred_element_type=jnp.float32)
    i = pl.program_id(0)

    @pl.when(i == 0)
    def _init():
        pooled_ref[...] = part

    @pl.when(i > 0)
    def _acc():
        pooled_ref[...] += part


_row = lambda i: (i, 0)
_whole = lambda i: (0, 0)
_HB = pl.BlockSpec((BLK, D), _row)
_AB = pl.BlockSpec((SC_CORES, BLK, D), lambda i: (0, i, 0))
_WB = pl.BlockSpec((D, D), _whole)
_VB = pl.BlockSpec((1, D), _whole)

_mlp_call = pl.pallas_call(
    _mlp_body,
    grid=(NB,),
    in_specs=[_HB, _AB, _WB, _VB, _VB, _VB, _WB, _VB, _VB, _VB],
    out_specs=_HB,
    out_shape=jax.ShapeDtypeStruct((N, D), jnp.float32),
)

_last_call = pl.pallas_call(
    _last_body,
    grid=(NB,),
    in_specs=[_HB, _AB, _WB, _VB, _VB, _VB, _WB, _VB,
              pl.BlockSpec((D, 1), _whole),
              pl.BlockSpec((1, 1, BLK), lambda i: (i, 0, 0))],
    out_specs=pl.BlockSpec((G, 1), _whole),
    out_shape=jax.ShapeDtypeStruct((G, 1), jnp.float32),
)


def kernel(x, edge_index, batch, params):
    src = edge_index[0].astype(jnp.int32)
    dst = edge_index[1].astype(jnp.int32)
    emb = jnp.zeros((32, D), jnp.float32).at[:22].set(params['embed'])
    x3 = x.astype(jnp.int32).reshape(NB, 1, BLK)
    h = _embed_call(x3, emb)

    def v(p):
        return p.reshape(1, D)

    for l in range(NUM_LAYERS):
        agg2 = _agg_sc(h, src, dst)
        if l < NUM_LAYERS - 1:
            h = _mlp_call(h, agg2,
                          params[f'l{l}_W1'], v(params[f'l{l}_b1']),
                          v(params[f'l{l}_mlp_ln_g']), v(params[f'l{l}_mlp_ln_b']),
                          params[f'l{l}_W2'], v(params[f'l{l}_b2']),
                          v(params[f'l{l}_out_ln_g']), v(params[f'l{l}_out_ln_b']))
        else:
            b3 = batch.astype(jnp.int32).reshape(NB, 1, BLK)
            pooled = _last_call(h, agg2,
                                params[f'l{l}_W1'], v(params[f'l{l}_b1']),
                                v(params[f'l{l}_mlp_ln_g']), v(params[f'l{l}_mlp_ln_b']),
                                params[f'l{l}_W2'], v(params[f'l{l}_b2']),
                                params['agg_W'], b3)

    return pooled + params['agg_b']


# speedup vs baseline: 3.1152x; 1.0092x over previous
import functools

import jax
import jax.numpy as jnp
from jax import lax
from jax.experimental import pallas as pl
from jax.experimental.pallas import tpu as pltpu
from jax.experimental.pallas import tpu_sc as plsc

N = 10000
E = 320000
G = 256
D = 128
NUM_LAYERS = 5
BLK = 400
NB = N // BLK

SC_CORES = 2
SC_SUBC = 16
NTILES = SC_CORES * SC_SUBC
CHUNK = 128
NCHUNKS = E // CHUNK
PCHUNK = 640
NPAD = SC_SUBC * PCHUNK
ROWS_PER_TILE = PCHUNK
NBUF = 4

_sc_mesh = plsc.VectorSubcoreMesh(
    core_axis_name="c", subcore_axis_name="s",
    num_cores=SC_CORES, num_subcores=SC_SUBC)

_Z16 = functools.partial(jnp.zeros, (16,), jnp.float32)

_PIECES = ((0, 128), (128, 128), (256, 128), (384, 128), (512, 128))


@functools.partial(
    pl.kernel,
    out_type=jax.ShapeDtypeStruct((SC_CORES, NPAD, D), jnp.float32),
    mesh=_sc_mesh,
    scratch_types=[
        pltpu.VMEM_SHARED((NPAD, D), jnp.float32),
        pltpu.VMEM((2, CHUNK), jnp.int32),
        pltpu.VMEM((2, CHUNK), jnp.int32),
        pltpu.VMEM((2, CHUNK, D), jnp.float32),
        pltpu.SemaphoreType.DMA,
        pltpu.SemaphoreType.DMA,
        pltpu.SemaphoreType.DMA,
    ],
)
def _agg_sc(h_hbm, src_hbm, dst_hbm, out_hbm, acc, sidx, didx, rows, g0, g1, s0):
    cid = lax.axis_index("c")
    sid = lax.axis_index("s")
    w = cid * SC_SUBC + sid

    def _zrow(i, carry):
        for j in range(8):
            rows[0, i, 16 * j:16 * j + 16] = _Z16()
        return carry

    lax.fori_loop(0, CHUNK, _zrow, None)
    off = sid * ROWS_PER_TILE
    for k, n in _PIECES:
        pltpu.sync_copy(rows.at[0, pl.ds(0, n)], acc.at[pl.ds(off + k, n)])
    plsc.subcore_barrier()

    npairs = NCHUNKS // 2
    rem = npairs % NTILES
    nc = jnp.where(w < rem, npairs // NTILES + 1, npairs // NTILES)

    def body(t, carry):
        base = (2 * w + 2 * NTILES * t) * CHUNK
        pltpu.sync_copy(src_hbm.at[pl.ds(base, CHUNK)], sidx.at[0])
        pltpu.async_copy(h_hbm.at[sidx.at[0]], rows.at[0], g0)
        pltpu.sync_copy(src_hbm.at[pl.ds(base + CHUNK, CHUNK)], sidx.at[1])
        pltpu.async_copy(h_hbm.at[sidx.at[1]], rows.at[1], g1)
        pltpu.sync_copy(dst_hbm.at[pl.ds(base, CHUNK)], didx.at[0])
        pltpu.sync_copy(dst_hbm.at[pl.ds(base + CHUNK, CHUNK)], didx.at[1])
        pltpu.make_async_copy(h_hbm.at[pl.ds(0, CHUNK)], rows.at[0], g0).wait()
        pltpu.async_copy(rows.at[0], acc.at[didx.at[0]], s0, add=True)
        pltpu.make_async_copy(h_hbm.at[pl.ds(0, CHUNK)], rows.at[1], g1).wait()
        pltpu.sync_copy(rows.at[1], acc.at[didx.at[1]], add=True)
        pltpu.make_async_copy(h_hbm.at[pl.ds(0, CHUNK)], acc.at[pl.ds(0, CHUNK)], s0).wait()
        return carry

    lax.fori_loop(0, nc, body, None)
    plsc.subcore_barrier()

    for k, n in _PIECES:
        pltpu.sync_copy(acc.at[pl.ds(off + k, n)], rows.at[0, pl.ds(0, n)])
        pltpu.sync_copy(rows.at[0, pl.ds(0, n)], out_hbm.at[cid, pl.ds(off + k, n)])


def _ln(z, g, b):
    mu = jnp.mean(z, axis=-1, keepdims=True)
    d = z - mu
    var = jnp.mean(d * d, axis=-1, keepdims=True)
    return d * lax.rsqrt(var + 1e-5) * g + b


def _embed_body(x_ref, emb_ref, out_ref):
    xb = x_ref[0, 0, :]
    ids = lax.broadcasted_iota(jnp.int32, (BLK, 32), 1)
    oh = (xb[:, None] == ids).astype(jnp.float32)
    out_ref[...] = jnp.dot(oh, emb_ref[...], preferred_element_type=jnp.float32)


_embed_call = pl.pallas_call(
    _embed_body,
    grid=(NB,),
    in_specs=[
        pl.BlockSpec((1, 1, BLK), lambda i: (i, 0, 0)),
        pl.BlockSpec((32, D), lambda i: (0, 0)),
    ],
    out_specs=pl.BlockSpec((BLK, D), lambda i: (i, 0)),
    out_shape=jax.ShapeDtypeStruct((N, D), jnp.float32),
)


def _mlp_body(h_ref, a_ref, w1_ref, b1_ref, g1_ref, e1_ref,
              w2_ref, b2_ref, g2_ref, e2_ref, out_ref):
    z = h_ref[...] + a_ref[0] + a_ref[1]
    z = jnp.dot(z, w1_ref[...], preferred_element_type=jnp.float32) + b1_ref[...]
    z = _ln(z, g1_ref[...], e1_ref[...])
    z = jnp.maximum(z, 0.0)
    h2 = jnp.dot(z, w2_ref[...], preferred_element_type=jnp.float32) + b2_ref[...]
    h2 = _ln(h2, g2_ref[...], e2_ref[...])
    out_ref[...] = jnp.maximum(h2, 0.0)


def _last_body(h_ref, a_ref, w1_ref, b1_ref, g1_ref, e1_ref,
               w2_ref, b2_ref, aggw_ref, batch_ref, pooled_ref):
    z = h_ref[...] + a_ref[0] + a_ref[1]
    z = jnp.dot(z, w1_ref[...], preferred_element_type=jnp.float32) + b1_ref[...]
    z = _ln(z, g1_ref[...], e1_ref[...])
    z = jnp.maximum(z, 0.0)
    h2 = jnp.dot(z, w2_ref[...], preferred_element_type=jnp.float32) + b2_ref[...]
    y = jnp.dot(h2, aggw_ref[...], preferred_element_type=jnp.float32)
    bb = batch_ref[0, 0, :]
    gids = lax.broadcasted_iota(jnp.int32, (G, BLK), 0)
    ohT = (gids == bb[None, :]).astype(jnp.float32)
    part = jnp.dot(ohT, y, preferred_element_type=jnp.float32)
    i = pl.program_id(0)

    @pl.when(i == 0)
    def _init():
        pooled_ref[...] = part

    @pl.when(i > 0)
    def _acc():
        pooled_ref[...] += part


_row = lambda i: (i, 0)
_whole = lambda i: (0, 0)
_HB = pl.BlockSpec((BLK, D), _row)
_AB = pl.BlockSpec((SC_CORES, BLK, D), lambda i: (0, i, 0))
_WB = pl.BlockSpec((D, D), _whole)
_VB = pl.BlockSpec((1, D), _whole)

_mlp_call = pl.pallas_call(
    _mlp_body,
    grid=(NB,),
    in_specs=[_HB, _AB, _WB, _VB, _VB, _VB, _WB, _VB, _VB, _VB],
    out_specs=_HB,
    out_shape=jax.ShapeDtypeStruct((N, D), jnp.float32),
)

_last_call = pl.pallas_call(
    _last_body,
    grid=(NB,),
    in_specs=[_HB, _AB, _WB, _VB, _VB, _VB, _WB, _VB,
              pl.BlockSpec((D, 1), _whole),
              pl.BlockSpec((1, 1, BLK), lambda i: (i, 0, 0))],
    out_specs=pl.BlockSpec((G, 1), _whole),
    out_shape=jax.ShapeDtypeStruct((G, 1), jnp.float32),
)


def kernel(x, edge_index, batch, params):
    src = edge_index[0].astype(jnp.int32)
    dst = edge_index[1].astype(jnp.int32)
    emb = jnp.zeros((32, D), jnp.float32).at[:22].set(params['embed'])
    x3 = x.astype(jnp.int32).reshape(NB, 1, BLK)
    h = _embed_call(x3, emb)

    def v(p):
        return p.reshape(1, D)

    for l in range(NUM_LAYERS):
        agg2 = _agg_sc(h, src, dst)
        if l < NUM_LAYERS - 1:
            h = _mlp_call(h, agg2,
                          params[f'l{l}_W1'], v(params[f'l{l}_b1']),
                          v(params[f'l{l}_mlp_ln_g']), v(params[f'l{l}_mlp_ln_b']),
                          params[f'l{l}_W2'], v(params[f'l{l}_b2']),
                          v(params[f'l{l}_out_ln_g']), v(params[f'l{l}_out_ln_b']))
        else:
            b3 = batch.astype(jnp.int32).reshape(NB, 1, BLK)
            pooled = _last_call(h, agg2,
                                params[f'l{l}_W1'], v(params[f'l{l}_b1']),
                                v(params[f'l{l}_mlp_ln_g']), v(params[f'l{l}_mlp_ln_b']),
                                params[f'l{l}_W2'], v(params[f'l{l}_b2']),
                                params['agg_W'], b3)

    return pooled + params['agg_b']
